# baseline (speedup 1.0000x reference)
import functools

import jax
import jax.numpy as jnp
from jax import lax
from jax.experimental import pallas as pl
from jax.experimental.pallas import tpu as pltpu
from jax.experimental.pallas import tpu_sc as plsc

VOCAB = 1000000
D_MODEL = 64
PAD_IDX = 0
GRP = 128
GRP_PER_CHUNK = 5


def _sc_gather(table, idx3d, n_rows, nw):
  g_per_w = idx3d.shape[1]
  chunks = g_per_w // GRP_PER_CHUNK
  rows_chunk = GRP * GRP_PER_CHUNK
  mesh = plsc.VectorSubcoreMesh(core_axis_name="c", subcore_axis_name="s")

  @functools.partial(
      pl.kernel,
      out_type=jax.ShapeDtypeStruct((n_rows, D_MODEL), jnp.float32),
      mesh=mesh,
      scratch_types=[
          pltpu.VMEM((g_per_w, GRP), jnp.int32),
          pltpu.VMEM((rows_chunk, D_MODEL), jnp.float32),
          pltpu.VMEM((rows_chunk, D_MODEL), jnp.float32),
          pltpu.SemaphoreType.DMA,
          pltpu.SemaphoreType.DMA,
      ],
      compiler_params=pltpu.CompilerParams(use_tc_tiling_on_sc=False),
  )
  def k(table_hbm, idx_hbm, out_hbm, idx_v, rows_a, rows_b, sem_a, sem_b):
    wid = lax.axis_index("s") * plsc.get_sparse_core_info().num_cores + \
        lax.axis_index("c")
    pltpu.sync_copy(idx_hbm.at[wid], idx_v)
    row_base = wid * g_per_w * GRP

    def gather_chunk(c, rows_v, sem):
      copies = []
      for j in range(GRP_PER_CHUNK):
        copies.append(pltpu.async_copy(
            table_hbm.at[idx_v.at[c * GRP_PER_CHUNK + j]],
            rows_v.at[pl.ds(j * GRP, GRP)], sem))
      return copies

    def drain_and_store(c, copies, rows_v):
      for cp in copies:
        cp.wait()
      pltpu.sync_copy(
          rows_v, out_hbm.at[pl.ds(row_base + c * rows_chunk, rows_chunk)])

    copies = gather_chunk(0, rows_a, sem_a)
    for c in range(chunks):
      buf = rows_a if c % 2 == 0 else rows_b
      nbuf, nsem = (rows_b, sem_b) if c % 2 == 0 else (rows_a, sem_a)
      ncopies = gather_chunk(c + 1, nbuf, nsem) if c + 1 < chunks else []
      drain_and_store(c, copies, buf)
      copies = ncopies

  return k(table, idx3d)


def _tc_finish_body(x_ref, ids_ref, role_ref, r_ref, o_ref):
  x = x_ref[0]
  ids = ids_ref[0]
  role = role_ref[0]
  keep = (ids != PAD_IDX)
  xm = jnp.where(keep, x, 0.0)
  t = lax.dot_general(xm, r_ref[...], (((1,), (1,)), ((), ())),
                      preferred_element_type=jnp.float32)
  o_ref[0] = jnp.where(role == 1, t, xm)


def _tc_finish(x_flat, ids_flat, role_flat, r, n_rows, bt):
  nb = n_rows // bt
  return pl.pallas_call(
      _tc_finish_body,
      out_shape=jax.ShapeDtypeStruct((nb, bt, D_MODEL), jnp.float32),
      grid=(nb,),
      in_specs=[
          pl.BlockSpec((1, bt, D_MODEL), lambda i: (i, 0, 0)),
          pl.BlockSpec((1, bt, 1), lambda i: (i, 0, 0)),
          pl.BlockSpec((1, bt, 1), lambda i: (i, 0, 0)),
          pl.BlockSpec((D_MODEL, D_MODEL), lambda i: (0, 0)),
      ],
      out_specs=pl.BlockSpec((1, bt, D_MODEL), lambda i: (i, 0, 0)),
  )(x_flat.reshape(nb, bt, D_MODEL), ids_flat.reshape(nb, bt, 1),
    role_flat.reshape(nb, bt, 1), r)


def kernel(input_ids, role_mask, table, R):
  b, l = input_ids.shape
  n = b * l
  ids_flat = input_ids.reshape(n).astype(jnp.int32)
  nw = 32
  idx3d = ids_flat.reshape(nw, n // (nw * GRP), GRP)
  x_flat = _sc_gather(table, idx3d, n, nw)
  bt = 2048
  out = _tc_finish(x_flat, ids_flat, role_mask.reshape(n).astype(jnp.int32),
                   R, n, bt)
  return out.reshape(b, l, D_MODEL)


# speedup vs baseline: 1.1044x; 1.1044x over previous
import functools

import jax
import jax.numpy as jnp
from jax import lax
from jax.experimental import pallas as pl
from jax.experimental.pallas import tpu as pltpu
from jax.experimental.pallas import tpu_sc as plsc

VOCAB = 1000000
D_MODEL = 64
PAD_IDX = 0
GRP = 128
GRP_PER_CHUNK = 5


def _sc_gather(table, idx3d, n_rows, nw):
  g_per_w = idx3d.shape[1]
  chunks = g_per_w // GRP_PER_CHUNK
  rows_chunk = GRP * GRP_PER_CHUNK
  mesh = plsc.VectorSubcoreMesh(core_axis_name="c", subcore_axis_name="s")

  @functools.partial(
      pl.kernel,
      out_type=jax.ShapeDtypeStruct((n_rows, D_MODEL), jnp.float32),
      mesh=mesh,
      scratch_types=[
          pltpu.VMEM((g_per_w, GRP), jnp.int32),
          pltpu.VMEM((rows_chunk, D_MODEL), jnp.float32),
          pltpu.VMEM((rows_chunk, D_MODEL), jnp.float32),
          pltpu.SemaphoreType.DMA,
          pltpu.SemaphoreType.DMA,
      ],
      compiler_params=pltpu.CompilerParams(use_tc_tiling_on_sc=False),
  )
  def k(table_hbm, idx_hbm, out_hbm, idx_v, rows_a, rows_b, sem_a, sem_b):
    wid = lax.axis_index("s") * plsc.get_sparse_core_info().num_cores + \
        lax.axis_index("c")
    pltpu.sync_copy(idx_hbm.at[wid], idx_v)
    row_base = wid * g_per_w * GRP

    def gather_chunk(c, rows_v, sem):
      copies = []
      for j in range(GRP_PER_CHUNK):
        copies.append(pltpu.async_copy(
            table_hbm.at[idx_v.at[c * GRP_PER_CHUNK + j]],
            rows_v.at[pl.ds(j * GRP, GRP)], sem))
      return copies

    def drain_and_store(c, copies, rows_v):
      for cp in copies:
        cp.wait()
      pltpu.sync_copy(
          rows_v, out_hbm.at[pl.ds(row_base + c * rows_chunk, rows_chunk)])

    copies = gather_chunk(0, rows_a, sem_a)
    for c in range(chunks):
      buf = rows_a if c % 2 == 0 else rows_b
      nbuf, nsem = (rows_b, sem_b) if c % 2 == 0 else (rows_a, sem_a)
      ncopies = gather_chunk(c + 1, nbuf, nsem) if c + 1 < chunks else []
      drain_and_store(c, copies, buf)
      copies = ncopies

  return k(table, idx3d)


def _tc_finish_body(x_ref, code_ref, r_ref, o_ref):
  x = x_ref[0]
  c8 = code_ref[0]
  ct = jnp.transpose(c8, (1, 0))
  c = ct[:, :1]
  xm = jnp.where(c != 0.0, x, 0.0)
  t = lax.dot_general(xm, r_ref[...], (((1,), (1,)), ((), ())),
                      preferred_element_type=jnp.float32)
  o_ref[0] = jnp.where(c == 2.0, t, xm)


def _tc_finish(x_flat, code8, r, n_rows, bt):
  nb = n_rows // bt
  return pl.pallas_call(
      _tc_finish_body,
      out_shape=jax.ShapeDtypeStruct((nb, bt, D_MODEL), jnp.float32),
      grid=(nb,),
      in_specs=[
          pl.BlockSpec((1, bt, D_MODEL), lambda i: (i, 0, 0)),
          pl.BlockSpec((1, 8, bt), lambda i: (i, 0, 0)),
          pl.BlockSpec((D_MODEL, D_MODEL), lambda i: (0, 0)),
      ],
      out_specs=pl.BlockSpec((1, bt, D_MODEL), lambda i: (i, 0, 0)),
  )(x_flat.reshape(nb, bt, D_MODEL), code8, r)


def kernel(input_ids, role_mask, table, R):
  b, l = input_ids.shape
  n = b * l
  ids_flat = input_ids.reshape(n).astype(jnp.int32)
  nw = 32
  idx3d = ids_flat.reshape(nw, n // (nw * GRP), GRP)
  x_flat = _sc_gather(table, idx3d, n, nw)
  bt = 2048
  nb = n // bt
  code = jnp.where(input_ids == PAD_IDX, 0.0,
                   1.0 + (role_mask == 1).astype(jnp.float32))
  code8 = jnp.broadcast_to(code.reshape(nb, 1, bt), (nb, 8, bt))
  out = _tc_finish(x_flat, code8, R, n, bt)
  return out.reshape(b, l, D_MODEL)


# speedup vs baseline: 1.2477x; 1.1298x over previous
import functools

import jax
import jax.numpy as jnp
from jax import lax
from jax.experimental import pallas as pl
from jax.experimental.pallas import tpu as pltpu
from jax.experimental.pallas import tpu_sc as plsc

VOCAB = 1000000
D_MODEL = 64
PAD_IDX = 0
GRP = 128
GRP_PER_CHUNK = 5
CHUNK = GRP * GRP_PER_CHUNK
HALF = CHUNK // 2
K_SUB = 8


def _sc_gather_packed(table, idx3d, n_rows, nw):
  g_per_w = idx3d.shape[1]
  chunks = g_per_w // GRP_PER_CHUNK
  rows_chunk = GRP * GRP_PER_CHUNK
  mesh = plsc.VectorSubcoreMesh(core_axis_name="c", subcore_axis_name="s")

  @functools.partial(
      pl.kernel,
      out_type=jax.ShapeDtypeStruct((n_rows // 2, 2 * D_MODEL), jnp.float32),
      mesh=mesh,
      scratch_types=[
          pltpu.VMEM((g_per_w, GRP), jnp.int32),
          pltpu.VMEM((rows_chunk, D_MODEL), jnp.float32),
          pltpu.VMEM((rows_chunk, D_MODEL), jnp.float32),
          pltpu.SemaphoreType.DMA,
          pltpu.SemaphoreType.DMA,
      ],
      compiler_params=pltpu.CompilerParams(use_tc_tiling_on_sc=False),
  )
  def k(table_hbm, idx_hbm, out_hbm, idx_v, rows_a, rows_b, sem_a, sem_b):
    wid = lax.axis_index("s") * plsc.get_sparse_core_info().num_cores + \
        lax.axis_index("c")
    pltpu.sync_copy(idx_hbm.at[wid], idx_v)
    row_base = wid * g_per_w * GRP // 2

    def gather_chunk(c, rows_v, sem):
      copies = []
      for j in range(GRP_PER_CHUNK):
        copies.append(pltpu.async_copy(
            table_hbm.at[idx_v.at[c * GRP_PER_CHUNK + j]],
            rows_v.at[pl.ds(j * GRP, GRP)], sem))
      return copies

    def drain_and_store(c, copies, rows_v):
      for cp in copies:
        cp.wait()
      p0 = row_base + c * HALF
      pltpu.sync_copy(rows_v.at[pl.ds(0, HALF)],
                      out_hbm.at[pl.ds(p0, HALF), pl.ds(0, D_MODEL)])
      pltpu.sync_copy(rows_v.at[pl.ds(HALF, HALF)],
                      out_hbm.at[pl.ds(p0, HALF), pl.ds(D_MODEL, D_MODEL)])

    copies = gather_chunk(0, rows_a, sem_a)
    for c in range(chunks):
      buf = rows_a if c % 2 == 0 else rows_b
      nbuf, nsem = (rows_b, sem_b) if c % 2 == 0 else (rows_a, sem_a)
      ncopies = gather_chunk(c + 1, nbuf, nsem) if c + 1 < chunks else []
      drain_and_store(c, copies, buf)
      copies = ncopies

  return k(table, idx3d)


def _tc_finish_body(x2_ref, code_ref, g_ref, o_ref):
  gmat = g_ref[...]
  for j in range(K_SUB):
    x2 = x2_ref[j]
    c8 = code_ref[j]
    ct = jnp.transpose(c8, (1, 0))
    ca = ct[:HALF, :1]
    cb = ct[HALF:, :1]
    lane = lax.broadcasted_iota(jnp.int32, (HALF, 2 * D_MODEL), 1)
    c2 = jnp.where(lane < D_MODEL, ca, cb)
    xm2 = jnp.where(c2 != 0.0, x2, 0.0)
    t2 = lax.dot_general(xm2, gmat, (((1,), (0,)), ((), ())),
                         preferred_element_type=jnp.float32)
    y2 = jnp.where(c2 == 2.0, t2, xm2)
    o_ref[j, :HALF] = y2[:, :D_MODEL]
    o_ref[j, HALF:] = y2[:, D_MODEL:]


def _tc_finish(x2, code8, g, n_rows):
  nc = n_rows // CHUNK
  grid = nc // K_SUB
  return pl.pallas_call(
      _tc_finish_body,
      out_shape=jax.ShapeDtypeStruct((nc, CHUNK, D_MODEL), jnp.float32),
      grid=(grid,),
      in_specs=[
          pl.BlockSpec((K_SUB, HALF, 2 * D_MODEL), lambda i: (i, 0, 0)),
          pl.BlockSpec((K_SUB, 8, CHUNK), lambda i: (i, 0, 0)),
          pl.BlockSpec((2 * D_MODEL, 2 * D_MODEL), lambda i: (0, 0)),
      ],
      out_specs=pl.BlockSpec((K_SUB, CHUNK, D_MODEL), lambda i: (i, 0, 0)),
  )(x2.reshape(nc, HALF, 2 * D_MODEL), code8, g)


def kernel(input_ids, role_mask, table, R):
  b, l = input_ids.shape
  n = b * l
  nc = n // CHUNK
  nw = 32
  ids_flat = input_ids.reshape(n).astype(jnp.int32)
  idx3d = ids_flat.reshape(nw, n // (nw * GRP), GRP)
  x2 = _sc_gather_packed(table, idx3d, n, nw)
  code = jnp.where(input_ids == PAD_IDX, 0.0,
                   1.0 + (role_mask == 1).astype(jnp.float32))
  code8 = jnp.broadcast_to(code.reshape(nc, 1, CHUNK), (nc, 8, CHUNK))
  zero = jnp.zeros((D_MODEL, D_MODEL), jnp.float32)
  g = jnp.block([[R.T, zero], [zero, R.T]])
  out = _tc_finish(x2, code8, g, n)
  return out.reshape(b, l, D_MODEL)
